# baseline (speedup 1.0000x reference)
import functools

import jax
import jax.numpy as jnp
from jax import lax
from jax.experimental import pallas as pl
from jax.experimental.pallas import tpu as pltpu
from jax.experimental.pallas import tpu_sc as plsc

_SLOPE = 0.01


def _leaky(v):
    return jnp.where(v >= 0, v, _SLOPE * v)


_CH = 128
_NB = 8
_PAD_ROWS = 8


def _sc_geometry(E):
    try:
        info = plsc.get_sparse_core_info()
        NC, NS = info.num_cores, info.num_subcores
    except ValueError:
        NC, NS = 2, 16
    NW = NC * NS
    NCH = -(-E // (NW * _CH))
    return NC, NS, NW, NCH


@functools.lru_cache(maxsize=None)
def _make_shift(N, E, D):
    NC, NS, NW, NCH = _sc_geometry(E)
    NP = N + _PAD_ROWS

    mesh = plsc.VectorSubcoreMesh(core_axis_name="c", subcore_axis_name="s",
                                  num_cores=NC, num_subcores=NS)

    scratch = [
        pltpu.VMEM((NCH, 2, _CH), jnp.int32),
        pltpu.VMEM((_NB, _CH, D), jnp.float32),
        pltpu.VMEM_SHARED((NP, D), jnp.float32),
        pltpu.SemaphoreType.DMA((_NB,)),
        pltpu.SemaphoreType.DMA((_NB,)),
    ]

    @functools.partial(
        pl.kernel,
        out_type=jax.ShapeDtypeStruct((NC, N, D), jnp.float32),
        mesh=mesh,
        scratch_types=scratch,
        compiler_params=pltpu.CompilerParams(use_tc_tiling_on_sc=False),
    )
    def shift(z_hbm, edges_hbm, zeros_hbm, part_hbm, idx, rows, acc,
              sem_g, sem_s):
        c = lax.axis_index("c")
        s = lax.axis_index("s")
        wid = c * NS + s

        pltpu.sync_copy(edges_hbm.at[pl.ds(wid * NCH, NCH)], idx)

        @pl.when(s == 0)
        def _():
            pltpu.sync_copy(zeros_hbm, acc)
        plsc.subcore_barrier()

        def start_gather(k, b):
            return pltpu.async_copy(z_hbm.at[idx.at[k, 0]], rows.at[b],
                                    sem_g.at[b])

        def start_scatter(k, b):
            return pltpu.async_copy(rows.at[b], acc.at[idx.at[k, 1]],
                                    sem_s.at[b], add=True)

        def wait_scatter(b):
            pltpu.make_async_copy(rows.at[b], acc.at[idx.at[0, 1]],
                                  sem_s.at[b]).wait()

        NG = NCH // _NB
        TAIL = NCH - NG * _NB

        def body(g, carry):
            descs = []
            for b in range(_NB):
                @pl.when(g > 0)
                def _(b=b):
                    wait_scatter(b)
                descs.append(start_gather(g * _NB + b, b))
            for b in range(_NB):
                descs[b].wait()
                start_scatter(g * _NB + b, b)
            return carry

        lax.fori_loop(0, NG, body, 0)

        tdescs = []
        for b in range(TAIL):
            if NG > 0:
                wait_scatter(b)
            tdescs.append(start_gather(NG * _NB + b, b))
        for b in range(TAIL):
            tdescs[b].wait()
            start_scatter(NG * _NB + b, b)
        for b in range(_NB):
            if b < TAIL or NG > 0:
                wait_scatter(b)

        plsc.subcore_barrier()

        @pl.when(s == 0)
        def _():
            pltpu.sync_copy(acc.at[pl.ds(0, N)], part_hbm.at[c])

    return shift


def _row_grid(M):
    for BM in (256, 128, 64, 8):
        if M % BM == 0:
            return M // BM, BM
    return M, 8


def _readin(x2, Wb, bb):
    M, K = x2.shape
    C = Wb.shape[1]
    G, BM = _row_grid(M)

    def body(x_ref, w_ref, b_ref, h_ref):
        h = jnp.dot(x_ref[...], w_ref[...],
                    preferred_element_type=jnp.float32) + b_ref[...]
        h_ref[...] = _leaky(h)

    return pl.pallas_call(
        body,
        grid=(G,),
        in_specs=[
            pl.BlockSpec((BM, K), lambda i: (i, 0)),
            pl.BlockSpec((K, C), lambda i: (0, 0)),
            pl.BlockSpec((1, C), lambda i: (0, 0)),
        ],
        out_specs=pl.BlockSpec((BM, C), lambda i: (i, 0)),
        out_shape=jax.ShapeDtypeStruct((M, C), jnp.float32),
    )(x2, Wb, bb)


def _pre(h4, Wb, bb):
    M, C = h4.shape
    G, BM = _row_grid(M)

    def body(h_ref, w_ref, b_ref, y_ref, o_ref):
        y = _leaky(h_ref[...])
        y_ref[...] = y
        o_ref[...] = jnp.dot(y, w_ref[...],
                             preferred_element_type=jnp.float32) + b_ref[...]

    return pl.pallas_call(
        body,
        grid=(G,),
        in_specs=[
            pl.BlockSpec((BM, C), lambda i: (i, 0)),
            pl.BlockSpec((C, C), lambda i: (0, 0)),
            pl.BlockSpec((1, C), lambda i: (0, 0)),
        ],
        out_specs=[
            pl.BlockSpec((BM, C), lambda i: (i, 0)),
            pl.BlockSpec((BM, C), lambda i: (i, 0)),
        ],
        out_shape=[
            jax.ShapeDtypeStruct((M, C), jnp.float32),
            jax.ShapeDtypeStruct((M, C), jnp.float32),
        ],
    )(h4, Wb, bb)


def _tap(p4, Wb, bb, out_in):
    _, M, C = p4.shape
    G, BM = _row_grid(M)

    def body(p_ref, w_ref, b_ref, oin_ref, z_ref, o_ref):
        z = p_ref[0] + p_ref[1]
        z_ref[...] = z
        o_ref[...] = oin_ref[...] + jnp.dot(
            z, w_ref[...], preferred_element_type=jnp.float32) + b_ref[...]

    return pl.pallas_call(
        body,
        grid=(G,),
        in_specs=[
            pl.BlockSpec((2, BM, C), lambda i: (0, i, 0)),
            pl.BlockSpec((C, C), lambda i: (0, 0)),
            pl.BlockSpec((1, C), lambda i: (0, 0)),
            pl.BlockSpec((BM, C), lambda i: (i, 0)),
        ],
        out_specs=[
            pl.BlockSpec((BM, C), lambda i: (i, 0)),
            pl.BlockSpec((BM, C), lambda i: (i, 0)),
        ],
        out_shape=[
            jax.ShapeDtypeStruct((M, C), jnp.float32),
            jax.ShapeDtypeStruct((M, C), jnp.float32),
        ],
    )(p4, Wb, bb, out_in)


def _last(p4, Wb, bb, out_in, h4):
    _, M, C = p4.shape
    G, BM = _row_grid(M)

    def body(p_ref, w_ref, b_ref, oin_ref, h_ref, hn_ref):
        z = p_ref[0] + p_ref[1]
        hn_ref[...] = h_ref[...] + oin_ref[...] + jnp.dot(
            z, w_ref[...], preferred_element_type=jnp.float32) + b_ref[...]

    return pl.pallas_call(
        body,
        grid=(G,),
        in_specs=[
            pl.BlockSpec((2, BM, C), lambda i: (0, i, 0)),
            pl.BlockSpec((C, C), lambda i: (0, 0)),
            pl.BlockSpec((1, C), lambda i: (0, 0)),
            pl.BlockSpec((BM, C), lambda i: (i, 0)),
            pl.BlockSpec((BM, C), lambda i: (i, 0)),
        ],
        out_specs=pl.BlockSpec((BM, C), lambda i: (i, 0)),
        out_shape=jax.ShapeDtypeStruct((M, C), jnp.float32),
    )(p4, Wb, bb, out_in, h4)


def _readout(h4, Wb, bb):
    M, C = h4.shape
    K = Wb.shape[1]
    G, BM = _row_grid(M)

    def body(h_ref, w_ref, b_ref, o_ref):
        o_ref[...] = jnp.dot(h_ref[...], w_ref[...],
                             preferred_element_type=jnp.float32) + b_ref[...]

    return pl.pallas_call(
        body,
        grid=(G,),
        in_specs=[
            pl.BlockSpec((BM, C), lambda i: (i, 0)),
            pl.BlockSpec((C, K), lambda i: (0, 0)),
            pl.BlockSpec((1, K), lambda i: (0, 0)),
        ],
        out_specs=pl.BlockSpec((BM, K), lambda i: (i, 0)),
        out_shape=jax.ShapeDtypeStruct((M, K), jnp.float32),
    )(h4, Wb, bb)


def kernel(x, edge_index, W_in, b_in, taps_W, taps_b, W_out, b_out):
    N, D_in = x.shape
    D = W_in.shape[1]
    D_out = W_out.shape[1]
    E = edge_index.shape[1]
    L, T1 = taps_W.shape[0], taps_W.shape[1]
    P = 128 // D
    assert P * D == 128 and N % P == 0
    M = N // P

    NC, NS, NW, NCH = _sc_geometry(E)
    E_pad = NW * NCH * _CH
    pad = E_pad - E
    if pad:
        ar = jnp.arange(pad, dtype=jnp.int32)
        src = jnp.concatenate([edge_index[0], ar % N])
        dst = jnp.concatenate([edge_index[1], N + (ar % _PAD_ROWS)])
    else:
        src, dst = edge_index[0], edge_index[1]
    edges3 = jnp.stack([src, dst]).reshape(2, NW * NCH, _CH).transpose(1, 0, 2)
    zeros = jnp.zeros((N + _PAD_ROWS, D), jnp.float32)
    shift = _make_shift(N, E, D)

    eyeP = jnp.eye(P, dtype=jnp.float32)
    Wb_in = jnp.kron(eyeP, W_in)
    bb_in = jnp.tile(b_in, P).reshape(1, 128)
    Wb_out = jnp.kron(eyeP, W_out)
    bb_out = jnp.tile(b_out, P).reshape(1, P * D_out)

    h4 = _readin(x.reshape(M, P * D_in), Wb_in, bb_in)
    for l in range(L):
        Wb = [jnp.kron(eyeP, taps_W[l, t]) for t in range(T1)]
        bb = [jnp.tile(taps_b[l, t], P).reshape(1, 128) for t in range(T1)]
        z4, out = _pre(h4, Wb[0], bb[0])
        for t in range(1, T1):
            p = shift(z4.reshape(N, D), edges3, zeros)
            p4 = p.reshape(NC, M, 128)
            if t < T1 - 1:
                z4, out = _tap(p4, Wb[t], bb[t], out)
            else:
                h4 = _last(p4, Wb[t], bb[t], out, h4)
    o4 = _readout(h4, Wb_out, bb_out)
    return o4.reshape(N, D_out)


# speedup vs baseline: 41.6347x; 41.6347x over previous
import functools

import jax
import jax.numpy as jnp
from jax import lax
from jax.experimental import pallas as pl
from jax.experimental.pallas import tpu as pltpu
from jax.experimental.pallas import tpu_sc as plsc

_SLOPE = 0.01


def _leaky(v):
    return jnp.where(v >= 0, v, _SLOPE * v)


_CH = 128
_NB = 8
_PAD_ROWS = 8


def _sc_geometry(E):
    try:
        info = plsc.get_sparse_core_info()
        NC, NS = info.num_cores, info.num_subcores
    except ValueError:
        NC, NS = 2, 16
    NW = NC * NS
    NCH = -(-E // (NW * _CH))
    return NC, NS, NW, NCH


@functools.lru_cache(maxsize=None)
def _make_shift(N, E, D):
    NC, NS, NW, NCH = _sc_geometry(E)
    NP = N + _PAD_ROWS

    mesh = plsc.VectorSubcoreMesh(core_axis_name="c", subcore_axis_name="s",
                                  num_cores=NC, num_subcores=NS)

    scratch = [
        pltpu.VMEM((NCH, 2, _CH), jnp.int32),
        pltpu.VMEM((_NB, _CH, D), jnp.float32),
        pltpu.VMEM_SHARED((NP, D), jnp.float32),
        pltpu.SemaphoreType.DMA((_NB,)),
        pltpu.SemaphoreType.DMA((_NB,)),
    ]

    @functools.partial(
        pl.kernel,
        out_type=jax.ShapeDtypeStruct((NC, N, D), jnp.float32),
        mesh=mesh,
        scratch_types=scratch,
        compiler_params=pltpu.CompilerParams(use_tc_tiling_on_sc=False),
    )
    def shift(z_hbm, edges_hbm, zeros_hbm, part_hbm, idx, rows, acc,
              sem_g, sem_s):
        c = lax.axis_index("c")
        s = lax.axis_index("s")
        wid = c * NS + s

        pltpu.sync_copy(edges_hbm.at[pl.ds(wid * NCH, NCH)], idx)

        @pl.when(s == 0)
        def _():
            pltpu.sync_copy(zeros_hbm, acc)
        plsc.subcore_barrier()

        def start_gather(k, b):
            return pltpu.async_copy(z_hbm.at[idx.at[k, 0]], rows.at[b],
                                    sem_g.at[b])

        def start_scatter(k, b):
            return pltpu.async_copy(rows.at[b], acc.at[idx.at[k, 1]],
                                    sem_s.at[b], add=True)

        def wait_scatter(b):
            pltpu.make_async_copy(rows.at[b], acc.at[idx.at[0, 1]],
                                  sem_s.at[b]).wait()

        NG = NCH // _NB
        TAIL = NCH - NG * _NB

        def body(g, carry):
            descs = []
            for b in range(_NB):
                @pl.when(g > 0)
                def _(b=b):
                    wait_scatter(b)
                descs.append(start_gather(g * _NB + b, b))
            for b in range(_NB):
                descs[b].wait()
                start_scatter(g * _NB + b, b)
            return carry

        lax.fori_loop(0, NG, body, 0)

        tdescs = []
        for b in range(TAIL):
            if NG > 0:
                wait_scatter(b)
            tdescs.append(start_gather(NG * _NB + b, b))
        for b in range(TAIL):
            tdescs[b].wait()
            start_scatter(NG * _NB + b, b)
        for b in range(_NB):
            if b < TAIL or NG > 0:
                wait_scatter(b)

        plsc.subcore_barrier()

        @pl.when(s == 0)
        def _():
            pltpu.sync_copy(acc.at[pl.ds(0, N)], part_hbm.at[c])

    return shift


def _row_grid(M):
    return 1, M


def _readin(x2, Wb, bb):
    M, K = x2.shape
    C = Wb.shape[1]
    G, BM = _row_grid(M)

    def body(x_ref, w_ref, b_ref, h_ref):
        h = jnp.dot(x_ref[...], w_ref[...],
                    preferred_element_type=jnp.float32) + b_ref[...]
        h_ref[...] = _leaky(h)

    return pl.pallas_call(
        body,
        grid=(G,),
        in_specs=[
            pl.BlockSpec((BM, K), lambda i: (i, 0)),
            pl.BlockSpec((K, C), lambda i: (0, 0)),
            pl.BlockSpec((1, C), lambda i: (0, 0)),
        ],
        out_specs=pl.BlockSpec((BM, C), lambda i: (i, 0)),
        out_shape=jax.ShapeDtypeStruct((M, C), jnp.float32),
    )(x2, Wb, bb)


def _pre(h4, Wb, bb):
    M, C = h4.shape
    G, BM = _row_grid(M)

    def body(h_ref, w_ref, b_ref, y_ref, o_ref):
        y = _leaky(h_ref[...])
        y_ref[...] = y
        o_ref[...] = jnp.dot(y, w_ref[...],
                             preferred_element_type=jnp.float32) + b_ref[...]

    return pl.pallas_call(
        body,
        grid=(G,),
        in_specs=[
            pl.BlockSpec((BM, C), lambda i: (i, 0)),
            pl.BlockSpec((C, C), lambda i: (0, 0)),
            pl.BlockSpec((1, C), lambda i: (0, 0)),
        ],
        out_specs=[
            pl.BlockSpec((BM, C), lambda i: (i, 0)),
            pl.BlockSpec((BM, C), lambda i: (i, 0)),
        ],
        out_shape=[
            jax.ShapeDtypeStruct((M, C), jnp.float32),
            jax.ShapeDtypeStruct((M, C), jnp.float32),
        ],
    )(h4, Wb, bb)


def _tap(p4, Wb, bb, out_in):
    _, M, C = p4.shape
    G, BM = _row_grid(M)

    def body(p_ref, w_ref, b_ref, oin_ref, z_ref, o_ref):
        z = p_ref[0] + p_ref[1]
        z_ref[...] = z
        o_ref[...] = oin_ref[...] + jnp.dot(
            z, w_ref[...], preferred_element_type=jnp.float32) + b_ref[...]

    return pl.pallas_call(
        body,
        grid=(G,),
        in_specs=[
            pl.BlockSpec((2, BM, C), lambda i: (0, i, 0)),
            pl.BlockSpec((C, C), lambda i: (0, 0)),
            pl.BlockSpec((1, C), lambda i: (0, 0)),
            pl.BlockSpec((BM, C), lambda i: (i, 0)),
        ],
        out_specs=[
            pl.BlockSpec((BM, C), lambda i: (i, 0)),
            pl.BlockSpec((BM, C), lambda i: (i, 0)),
        ],
        out_shape=[
            jax.ShapeDtypeStruct((M, C), jnp.float32),
            jax.ShapeDtypeStruct((M, C), jnp.float32),
        ],
    )(p4, Wb, bb, out_in)


def _last(p4, Wb, bb, out_in, h4):
    _, M, C = p4.shape
    G, BM = _row_grid(M)

    def body(p_ref, w_ref, b_ref, oin_ref, h_ref, hn_ref):
        z = p_ref[0] + p_ref[1]
        hn_ref[...] = h_ref[...] + oin_ref[...] + jnp.dot(
            z, w_ref[...], preferred_element_type=jnp.float32) + b_ref[...]

    return pl.pallas_call(
        body,
        grid=(G,),
        in_specs=[
            pl.BlockSpec((2, BM, C), lambda i: (0, i, 0)),
            pl.BlockSpec((C, C), lambda i: (0, 0)),
            pl.BlockSpec((1, C), lambda i: (0, 0)),
            pl.BlockSpec((BM, C), lambda i: (i, 0)),
            pl.BlockSpec((BM, C), lambda i: (i, 0)),
        ],
        out_specs=pl.BlockSpec((BM, C), lambda i: (i, 0)),
        out_shape=jax.ShapeDtypeStruct((M, C), jnp.float32),
    )(p4, Wb, bb, out_in, h4)


def _readout(h4, Wb, bb):
    M, C = h4.shape
    K = Wb.shape[1]
    G, BM = _row_grid(M)

    def body(h_ref, w_ref, b_ref, o_ref):
        o_ref[...] = jnp.dot(h_ref[...], w_ref[...],
                             preferred_element_type=jnp.float32) + b_ref[...]

    return pl.pallas_call(
        body,
        grid=(G,),
        in_specs=[
            pl.BlockSpec((BM, C), lambda i: (i, 0)),
            pl.BlockSpec((C, K), lambda i: (0, 0)),
            pl.BlockSpec((1, K), lambda i: (0, 0)),
        ],
        out_specs=pl.BlockSpec((BM, K), lambda i: (i, 0)),
        out_shape=jax.ShapeDtypeStruct((M, K), jnp.float32),
    )(h4, Wb, bb)


def kernel(x, edge_index, W_in, b_in, taps_W, taps_b, W_out, b_out):
    N, D_in = x.shape
    D = W_in.shape[1]
    D_out = W_out.shape[1]
    E = edge_index.shape[1]
    L, T1 = taps_W.shape[0], taps_W.shape[1]
    P = 128 // D
    assert P * D == 128 and N % P == 0
    M = N // P

    NC, NS, NW, NCH = _sc_geometry(E)
    E_pad = NW * NCH * _CH
    pad = E_pad - E
    if pad:
        ar = jnp.arange(pad, dtype=jnp.int32)
        src = jnp.concatenate([edge_index[0], ar % N])
        dst = jnp.concatenate([edge_index[1], N + (ar % _PAD_ROWS)])
    else:
        src, dst = edge_index[0], edge_index[1]
    edges3 = jnp.stack([src, dst]).reshape(2, NW * NCH, _CH).transpose(1, 0, 2)
    zeros = jnp.zeros((N + _PAD_ROWS, D), jnp.float32)
    shift = _make_shift(N, E, D)

    eyeP = jnp.eye(P, dtype=jnp.float32)
    Wb_in = jnp.kron(eyeP, W_in)
    bb_in = jnp.tile(b_in, P).reshape(1, 128)
    Wb_out = jnp.kron(eyeP, W_out)
    bb_out = jnp.tile(b_out, P).reshape(1, P * D_out)

    h4 = _readin(x.reshape(M, P * D_in), Wb_in, bb_in)
    for l in range(L):
        Wb = [jnp.kron(eyeP, taps_W[l, t]) for t in range(T1)]
        bb = [jnp.tile(taps_b[l, t], P).reshape(1, 128) for t in range(T1)]
        z4, out = _pre(h4, Wb[0], bb[0])
        for t in range(1, T1):
            p = shift(z4.reshape(N, D), edges3, zeros)
            p4 = p.reshape(NC, M, 128)
            if t < T1 - 1:
                z4, out = _tap(p4, Wb[t], bb[t], out)
            else:
                h4 = _last(p4, Wb[t], bb[t], out, h4)
    o4 = _readout(h4, Wb_out, bb_out)
    return o4.reshape(N, D_out)


# speedup vs baseline: 45.4379x; 1.0913x over previous
import functools

import jax
import jax.numpy as jnp
from jax import lax
from jax.experimental import pallas as pl
from jax.experimental.pallas import tpu as pltpu
from jax.experimental.pallas import tpu_sc as plsc

_SLOPE = 0.01


def _leaky(v):
    return jnp.where(v >= 0, v, _SLOPE * v)


_CH = 128
_NB = 8
_PAD_ROWS = 128


def _sc_geometry(E):
    try:
        info = plsc.get_sparse_core_info()
        NC, NS = info.num_cores, info.num_subcores
    except ValueError:
        NC, NS = 2, 16
    NW = NC * NS
    NCH = -(-E // (NW * _CH))
    NCH = -(-NCH // _NB) * _NB
    return NC, NS, NW, NCH


@functools.lru_cache(maxsize=None)
def _make_shift(N, E, D):
    NC, NS, NW, NCH = _sc_geometry(E)
    NP = N + _PAD_ROWS
    assert N % NS == 0 and NP % NS == 0
    RPT_W = N // NS
    RPT_Z = NP // NS

    mesh = plsc.VectorSubcoreMesh(core_axis_name="c", subcore_axis_name="s",
                                  num_cores=NC, num_subcores=NS)

    scratch = [
        pltpu.VMEM((NCH, 2, _CH), jnp.int32),
        pltpu.VMEM((_NB, _CH, D), jnp.float32),
        pltpu.VMEM_SHARED((NP, D), jnp.float32),
        pltpu.SemaphoreType.DMA((_NB,)),
        pltpu.SemaphoreType.DMA((_NB,)),
    ]

    @functools.partial(
        pl.kernel,
        out_type=jax.ShapeDtypeStruct((NC, N, D), jnp.float32),
        mesh=mesh,
        scratch_types=scratch,
        compiler_params=pltpu.CompilerParams(use_tc_tiling_on_sc=False),
    )
    def shift(z_hbm, edges_hbm, zeros_hbm, part_hbm, idx, rows, acc,
              sem_g, sem_s):
        c = lax.axis_index("c")
        s = lax.axis_index("s")
        wid = c * NS + s

        def start_gather(k, b):
            return pltpu.async_copy(z_hbm.at[idx.at[k, 0]], rows.at[b],
                                    sem_g.at[b])

        def start_scatter(k, b):
            return pltpu.async_copy(rows.at[b], acc.at[idx.at[k, 1]],
                                    sem_s.at[b], add=True)

        def wait_gather(b):
            pltpu.make_async_copy(z_hbm.at[idx.at[0, 0]], rows.at[b],
                                  sem_g.at[b]).wait()

        def wait_scatter(b):
            pltpu.make_async_copy(rows.at[b], acc.at[idx.at[0, 1]],
                                  sem_s.at[b]).wait()

        pltpu.sync_copy(edges_hbm.at[pl.ds(wid * NCH, NCH)], idx)
        for b in range(_NB):
            start_gather(b, b)
        pltpu.sync_copy(zeros_hbm.at[pl.ds(s * RPT_Z, RPT_Z)],
                        acc.at[pl.ds(s * RPT_Z, RPT_Z)])
        plsc.subcore_barrier()

        NG = NCH // _NB

        def body(g, carry):
            for b in range(_NB):
                wait_gather(b)
                start_scatter(g * _NB + b, b)
            for b in range(_NB):
                @pl.when(g < NG - 1)
                def _(b=b):
                    wait_scatter(b)
                    start_gather((g + 1) * _NB + b, b)
            return carry

        lax.fori_loop(0, NG, body, 0)
        for b in range(_NB):
            wait_scatter(b)

        plsc.subcore_barrier()
        pltpu.sync_copy(acc.at[pl.ds(s * RPT_W, RPT_W)],
                        part_hbm.at[c, pl.ds(s * RPT_W, RPT_W)])

    return shift


def _row_grid(M):
    return 1, M


def _readin(x2, Wb, bb):
    M, K = x2.shape
    C = Wb.shape[1]
    G, BM = _row_grid(M)

    def body(x_ref, w_ref, b_ref, h_ref):
        h = jnp.dot(x_ref[...], w_ref[...],
                    preferred_element_type=jnp.float32) + b_ref[...]
        h_ref[...] = _leaky(h)

    return pl.pallas_call(
        body,
        grid=(G,),
        in_specs=[
            pl.BlockSpec((BM, K), lambda i: (i, 0)),
            pl.BlockSpec((K, C), lambda i: (0, 0)),
            pl.BlockSpec((1, C), lambda i: (0, 0)),
        ],
        out_specs=pl.BlockSpec((BM, C), lambda i: (i, 0)),
        out_shape=jax.ShapeDtypeStruct((M, C), jnp.float32),
    )(x2, Wb, bb)


def _pre(h4, Wb, bb):
    M, C = h4.shape
    G, BM = _row_grid(M)

    def body(h_ref, w_ref, b_ref, y_ref, o_ref):
        y = _leaky(h_ref[...])
        y_ref[...] = y
        o_ref[...] = jnp.dot(y, w_ref[...],
                             preferred_element_type=jnp.float32) + b_ref[...]

    return pl.pallas_call(
        body,
        grid=(G,),
        in_specs=[
            pl.BlockSpec((BM, C), lambda i: (i, 0)),
            pl.BlockSpec((C, C), lambda i: (0, 0)),
            pl.BlockSpec((1, C), lambda i: (0, 0)),
        ],
        out_specs=[
            pl.BlockSpec((BM, C), lambda i: (i, 0)),
            pl.BlockSpec((BM, C), lambda i: (i, 0)),
        ],
        out_shape=[
            jax.ShapeDtypeStruct((M, C), jnp.float32),
            jax.ShapeDtypeStruct((M, C), jnp.float32),
        ],
    )(h4, Wb, bb)


def _tap(p4, Wb, bb, out_in):
    _, M, C = p4.shape
    G, BM = _row_grid(M)

    def body(p_ref, w_ref, b_ref, oin_ref, z_ref, o_ref):
        z = p_ref[0] + p_ref[1]
        z_ref[...] = z
        o_ref[...] = oin_ref[...] + jnp.dot(
            z, w_ref[...], preferred_element_type=jnp.float32) + b_ref[...]

    return pl.pallas_call(
        body,
        grid=(G,),
        in_specs=[
            pl.BlockSpec((2, BM, C), lambda i: (0, i, 0)),
            pl.BlockSpec((C, C), lambda i: (0, 0)),
            pl.BlockSpec((1, C), lambda i: (0, 0)),
            pl.BlockSpec((BM, C), lambda i: (i, 0)),
        ],
        out_specs=[
            pl.BlockSpec((BM, C), lambda i: (i, 0)),
            pl.BlockSpec((BM, C), lambda i: (i, 0)),
        ],
        out_shape=[
            jax.ShapeDtypeStruct((M, C), jnp.float32),
            jax.ShapeDtypeStruct((M, C), jnp.float32),
        ],
    )(p4, Wb, bb, out_in)


def _last(p4, Wb, bb, out_in, h4):
    _, M, C = p4.shape
    G, BM = _row_grid(M)

    def body(p_ref, w_ref, b_ref, oin_ref, h_ref, hn_ref):
        z = p_ref[0] + p_ref[1]
        hn_ref[...] = h_ref[...] + oin_ref[...] + jnp.dot(
            z, w_ref[...], preferred_element_type=jnp.float32) + b_ref[...]

    return pl.pallas_call(
        body,
        grid=(G,),
        in_specs=[
            pl.BlockSpec((2, BM, C), lambda i: (0, i, 0)),
            pl.BlockSpec((C, C), lambda i: (0, 0)),
            pl.BlockSpec((1, C), lambda i: (0, 0)),
            pl.BlockSpec((BM, C), lambda i: (i, 0)),
            pl.BlockSpec((BM, C), lambda i: (i, 0)),
        ],
        out_specs=pl.BlockSpec((BM, C), lambda i: (i, 0)),
        out_shape=jax.ShapeDtypeStruct((M, C), jnp.float32),
    )(p4, Wb, bb, out_in, h4)


def _readout(h4, Wb, bb):
    M, C = h4.shape
    K = Wb.shape[1]
    G, BM = _row_grid(M)

    def body(h_ref, w_ref, b_ref, o_ref):
        o_ref[...] = jnp.dot(h_ref[...], w_ref[...],
                             preferred_element_type=jnp.float32) + b_ref[...]

    return pl.pallas_call(
        body,
        grid=(G,),
        in_specs=[
            pl.BlockSpec((BM, C), lambda i: (i, 0)),
            pl.BlockSpec((C, K), lambda i: (0, 0)),
            pl.BlockSpec((1, K), lambda i: (0, 0)),
        ],
        out_specs=pl.BlockSpec((BM, K), lambda i: (i, 0)),
        out_shape=jax.ShapeDtypeStruct((M, K), jnp.float32),
    )(h4, Wb, bb)


def kernel(x, edge_index, W_in, b_in, taps_W, taps_b, W_out, b_out):
    N, D_in = x.shape
    D = W_in.shape[1]
    D_out = W_out.shape[1]
    E = edge_index.shape[1]
    L, T1 = taps_W.shape[0], taps_W.shape[1]
    P = 128 // D
    assert P * D == 128
    Np = -(-N // 32) * 32
    M = Np // P

    NC, NS, NW, NCH = _sc_geometry(E)
    E_pad = NW * NCH * _CH
    pad = E_pad - E
    if pad:
        ar = jnp.arange(pad, dtype=jnp.int32)
        src = jnp.concatenate([edge_index[0], ar % N])
        dst = jnp.concatenate([edge_index[1], Np + (ar % _PAD_ROWS)])
    else:
        src, dst = edge_index[0], edge_index[1]
    edges3 = jnp.stack([src, dst]).reshape(2, NW * NCH, _CH).transpose(1, 0, 2)
    zeros = jnp.zeros((Np + _PAD_ROWS, D), jnp.float32)
    shift = _make_shift(Np, E, D)
    if Np > N:
        x = jnp.pad(x, ((0, Np - N), (0, 0)))

    eyeP = jnp.eye(P, dtype=jnp.float32)
    Wb_in = jnp.kron(eyeP, W_in)
    bb_in = jnp.tile(b_in, P).reshape(1, 128)
    Wb_out = jnp.kron(eyeP, W_out)
    bb_out = jnp.tile(b_out, P).reshape(1, P * D_out)

    h4 = _readin(x.reshape(M, P * D_in), Wb_in, bb_in)
    for l in range(L):
        Wb = [jnp.kron(eyeP, taps_W[l, t]) for t in range(T1)]
        bb = [jnp.tile(taps_b[l, t], P).reshape(1, 128) for t in range(T1)]
        z4, out = _pre(h4, Wb[0], bb[0])
        for t in range(1, T1):
            p = shift(z4.reshape(Np, D), edges3, zeros)
            p4 = p.reshape(NC, M, 128)
            if t < T1 - 1:
                z4, out = _tap(p4, Wb[t], bb[t], out)
            else:
                h4 = _last(p4, Wb[t], bb[t], out, h4)
    o4 = _readout(h4, Wb_out, bb_out)
    return o4.reshape(Np, D_out)[:N]
